# Initial kernel scaffold; baseline (speedup 1.0000x reference)
import jax
import jax.numpy as jnp
from jax.experimental import pallas as pl
from jax.experimental.pallas import tpu as pltpu

NUM_FEATURES = 2048
NUM_SAMPLES = 8
SINKHORN_NUM_ITERS = 20
INV_TEMP = 10.0


def _sinkhorn_kernel(noise_ref, gamma_ref, out_ref):
    x = (gamma_ref[...] + noise_ref[0]) * INV_TEMP
    for _ in range(SINKHORN_NUM_ITERS):
        m = jnp.max(x, axis=1, keepdims=True)
        x = x - (m + jnp.log(jnp.sum(jnp.exp(x - m), axis=1, keepdims=True)))
        m = jnp.max(x, axis=0, keepdims=True)
        x = x - (m + jnp.log(jnp.sum(jnp.exp(x - m), axis=0, keepdims=True)))
    out_ref[0] = jnp.exp(x)


def kernel(gumbel_noise, gamma):
    n = NUM_FEATURES
    return pl.pallas_call(
        _sinkhorn_kernel,
        grid=(NUM_SAMPLES,),
        in_specs=[
            pl.BlockSpec((1, n, n), lambda i: (i, 0, 0)),
            pl.BlockSpec((n, n), lambda i: (0, 0)),
        ],
        out_specs=pl.BlockSpec((1, n, n), lambda i: (i, 0, 0)),
        out_shape=jax.ShapeDtypeStruct((NUM_SAMPLES, n, n), jnp.float32),
        compiler_params=pltpu.CompilerParams(
            dimension_semantics=("arbitrary",),
        ),
    )(gumbel_noise, gamma)


# speedup vs baseline: 5.3650x; 5.3650x over previous
import jax
import jax.numpy as jnp
from jax import lax
from jax.experimental import pallas as pl
from jax.experimental.pallas import tpu as pltpu

N = 2048
S = 8
ITERS = 20
INV_TEMP = 10.0
CH = 256
NCH = N // CH


def _slog(v):
    return jnp.log(jnp.maximum(v, 1e-37))


def _sinkhorn_kernel(noise_hbm, gamma_hbm, out_hbm, x, g,
                     sem_in, sem_g, sem_out):
    i = pl.program_id(0)

    cp_g = pltpu.make_async_copy(gamma_hbm, g, sem_g)
    cp_g.start()
    cp_in = pltpu.make_async_copy(noise_hbm.at[i], x, sem_in)
    cp_in.start()
    cp_g.wait()
    cp_in.wait()

    def chunk1(k, s):
        rows = pl.ds(k * CH, CH)
        c = (x[rows, :] + g[rows, :]) * INV_TEMP
        m = jnp.max(c, axis=1, keepdims=True)
        e = jnp.exp(c - m)
        rs = jnp.sum(e, axis=1, keepdims=True)
        x[rows, :] = c - (m + _slog(rs))
        return s + jnp.sum(e * (1.0 / rs), axis=0, keepdims=True)

    s = lax.fori_loop(0, NCH, chunk1, jnp.zeros((1, N), jnp.float32))
    co = _slog(s)

    def sweep(_, co):
        def chunk(k, s):
            rows = pl.ds(k * CH, CH)
            c = x[rows, :] - co
            e = jnp.exp(c)
            rs = jnp.sum(e, axis=1, keepdims=True)
            x[rows, :] = c - _slog(rs)
            return s + jnp.sum(e * (1.0 / rs), axis=0, keepdims=True)

        s = lax.fori_loop(0, NCH, chunk, jnp.zeros((1, N), jnp.float32))
        return _slog(s)

    co = lax.fori_loop(0, ITERS - 1, sweep, co)

    def finalize(k, _):
        rows = pl.ds(k * CH, CH)
        x[rows, :] = jnp.exp(x[rows, :] - co)
        return 0

    lax.fori_loop(0, NCH, finalize, 0)

    cp_out = pltpu.make_async_copy(x, out_hbm.at[i], sem_out)
    cp_out.start()
    cp_out.wait()


def kernel(gumbel_noise, gamma):
    return pl.pallas_call(
        _sinkhorn_kernel,
        grid=(S,),
        in_specs=[
            pl.BlockSpec(memory_space=pltpu.MemorySpace.HBM),
            pl.BlockSpec(memory_space=pltpu.MemorySpace.HBM),
        ],
        out_specs=pl.BlockSpec(memory_space=pltpu.MemorySpace.HBM),
        out_shape=jax.ShapeDtypeStruct((S, N, N), jnp.float32),
        scratch_shapes=[
            pltpu.VMEM((N, N), jnp.float32),
            pltpu.VMEM((N, N), jnp.float32),
            pltpu.SemaphoreType.DMA,
            pltpu.SemaphoreType.DMA,
            pltpu.SemaphoreType.DMA,
        ],
        compiler_params=pltpu.CompilerParams(
            dimension_semantics=("arbitrary",),
        ),
    )(gumbel_noise, gamma)


# speedup vs baseline: 9.5640x; 1.7827x over previous
import jax
import jax.numpy as jnp
from jax import lax
from jax.experimental import pallas as pl
from jax.experimental.pallas import tpu as pltpu

N = 2048
S = 8
ITERS = 20
INV_TEMP = 10.0
CH = 256
NCH = N // CH
TINY = 1e-37


def _sinkhorn_kernel(noise_hbm, gamma_hbm, out_hbm, x, g,
                     sem_in, sem_g, sem_out):
    i = pl.program_id(0)

    cp_g = pltpu.make_async_copy(gamma_hbm, g, sem_g)
    cp_g.start()
    cp_in = pltpu.make_async_copy(noise_hbm.at[i], x, sem_in)
    cp_in.start()
    cp_g.wait()
    cp_in.wait()

    def chunk1(k, s):
        rows = pl.ds(k * CH, CH)
        c = (x[rows, :] + g[rows, :]) * INV_TEMP
        m = jnp.max(c, axis=1, keepdims=True)
        e = jnp.exp(c - m)
        rs = jnp.sum(e, axis=1, keepdims=True)
        p = e * (1.0 / rs)
        x[rows, :] = p
        return s + jnp.sum(p, axis=0, keepdims=True)

    s = lax.fori_loop(0, NCH, chunk1, jnp.zeros((1, N), jnp.float32))
    b = 1.0 / jnp.maximum(s, TINY)

    def sinkhorn_pass(_, carry):
        b, _b_old = carry

        def chunk(k, s):
            rows = pl.ds(k * CH, CH)
            q = x[rows, :] * b
            a = 1.0 / jnp.maximum(jnp.sum(q, axis=1, keepdims=True), TINY)
            return s + jnp.sum(q * a, axis=0, keepdims=True)

        acc = lax.fori_loop(0, NCH, chunk, jnp.zeros((1, N), jnp.float32))
        return b * (1.0 / jnp.maximum(acc, TINY)), b

    b, b_prev = lax.fori_loop(0, ITERS - 1, sinkhorn_pass, (b, b))

    beta = b * (1.0 / b_prev)

    def finalize(k, _):
        rows = pl.ds(k * CH, CH)
        q = x[rows, :] * b_prev
        a = 1.0 / jnp.maximum(jnp.sum(q, axis=1, keepdims=True), TINY)
        x[rows, :] = q * a * beta
        return 0

    lax.fori_loop(0, NCH, finalize, 0)

    cp_out = pltpu.make_async_copy(x, out_hbm.at[i], sem_out)
    cp_out.start()
    cp_out.wait()


def kernel(gumbel_noise, gamma):
    return pl.pallas_call(
        _sinkhorn_kernel,
        grid=(S,),
        in_specs=[
            pl.BlockSpec(memory_space=pltpu.MemorySpace.HBM),
            pl.BlockSpec(memory_space=pltpu.MemorySpace.HBM),
        ],
        out_specs=pl.BlockSpec(memory_space=pltpu.MemorySpace.HBM),
        out_shape=jax.ShapeDtypeStruct((S, N, N), jnp.float32),
        scratch_shapes=[
            pltpu.VMEM((N, N), jnp.float32),
            pltpu.VMEM((N, N), jnp.float32),
            pltpu.SemaphoreType.DMA,
            pltpu.SemaphoreType.DMA,
            pltpu.SemaphoreType.DMA,
        ],
        compiler_params=pltpu.CompilerParams(
            dimension_semantics=("arbitrary",),
        ),
    )(gumbel_noise, gamma)


# speedup vs baseline: 10.3939x; 1.0868x over previous
import jax
import jax.numpy as jnp
from jax import lax
from jax.experimental import pallas as pl
from jax.experimental.pallas import tpu as pltpu

N = 2048
S = 8
ITERS = 20
INV_TEMP = 10.0
CH = 256
NCH = N // CH
TINY = 1e-37


def _sinkhorn_kernel(noise_hbm, gamma_hbm, out_hbm, x, g,
                     sem_in, sem_g, sem_out):
    i = pl.program_id(0)

    cp_g = pltpu.make_async_copy(gamma_hbm, g, sem_g)
    cp_g.start()
    cp_in = pltpu.make_async_copy(noise_hbm.at[i], x, sem_in)
    cp_in.start()
    cp_g.wait()
    cp_in.wait()

    def chunk1(k, s):
        rows = pl.ds(k * CH, CH)
        c = (x[rows, :] + g[rows, :]) * INV_TEMP
        m = jnp.max(c, axis=1, keepdims=True)
        e = jnp.exp(c - m)
        rs = jnp.sum(e, axis=1, keepdims=True)
        p = e * (1.0 / rs)
        x[rows, :] = p
        return s + jnp.sum(p, axis=0, keepdims=True)

    s = lax.fori_loop(0, NCH, chunk1, jnp.zeros((1, N), jnp.float32))
    b = 1.0 / jnp.maximum(s, TINY)

    def sinkhorn_pass(_, carry):
        b, _b_old = carry

        def chunk(k, s):
            rows = pl.ds(k * CH, CH)
            r = jnp.sum(x[rows, :] * b, axis=1, keepdims=True)
            a = 1.0 / jnp.maximum(r, TINY)
            t = x[rows, :] * a
            return s + jnp.sum(t.reshape(CH // 8, 8, N), axis=0)

        acc8 = lax.fori_loop(0, NCH, chunk, jnp.zeros((8, N), jnp.float32))
        acc = jnp.sum(acc8, axis=0, keepdims=True)
        return 1.0 / jnp.maximum(acc, TINY), b

    b, b_prev = lax.fori_loop(0, ITERS - 1, sinkhorn_pass, (b, b))

    beta = b * (1.0 / b_prev)

    def finalize(k, _):
        rows = pl.ds(k * CH, CH)
        q = x[rows, :] * b_prev
        a = 1.0 / jnp.maximum(jnp.sum(q, axis=1, keepdims=True), TINY)
        x[rows, :] = q * a * beta
        return 0

    lax.fori_loop(0, NCH, finalize, 0)

    cp_out = pltpu.make_async_copy(x, out_hbm.at[i], sem_out)
    cp_out.start()
    cp_out.wait()


def kernel(gumbel_noise, gamma):
    return pl.pallas_call(
        _sinkhorn_kernel,
        grid=(S,),
        in_specs=[
            pl.BlockSpec(memory_space=pltpu.MemorySpace.HBM),
            pl.BlockSpec(memory_space=pltpu.MemorySpace.HBM),
        ],
        out_specs=pl.BlockSpec(memory_space=pltpu.MemorySpace.HBM),
        out_shape=jax.ShapeDtypeStruct((S, N, N), jnp.float32),
        scratch_shapes=[
            pltpu.VMEM((N, N), jnp.float32),
            pltpu.VMEM((N, N), jnp.float32),
            pltpu.SemaphoreType.DMA,
            pltpu.SemaphoreType.DMA,
            pltpu.SemaphoreType.DMA,
        ],
        compiler_params=pltpu.CompilerParams(
            dimension_semantics=("arbitrary",),
        ),
    )(gumbel_noise, gamma)
